# Initial kernel scaffold; baseline (speedup 1.0000x reference)
import jax
import jax.numpy as jnp
from jax.experimental import pallas as pl


def kernel(prop_coords, prop_feats, orig_coords, orig_feats, W0, b0, g0, be0, W1, b1, g1, be1):
    raise NotImplementedError("write your pallas kernel here")


# speedup vs baseline: 18.3172x; 18.3172x over previous
import jax
import jax.numpy as jnp
from jax.experimental import pallas as pl

_HIGHEST = jax.lax.Precision.HIGHEST
_EPS = 1e-5


def _pass_a(q_ref, of_ref, pt_ref, pf_ref, w0c_ref, w0f_ref, w0i_ref, b0_ref,
            y0_ref, st_ref):
    i = pl.program_id(0)
    q = q_ref[...]
    pt = pt_ref[...]
    s = jnp.dot(q, pt, precision=_HIGHEST, preferred_element_type=jnp.float32)
    qn = jnp.sum(q * q, axis=1, keepdims=True)
    pn = jnp.sum(pt * pt, axis=0, keepdims=True)
    d2 = (qn - 2.0 * s) + pn
    m1 = jnp.min(d2, axis=1, keepdims=True)
    t = jnp.where(d2 == m1, jnp.inf, d2)
    m2 = jnp.min(t, axis=1, keepdims=True)
    t = jnp.where(t == m2, jnp.inf, t)
    m3 = jnp.min(t, axis=1, keepdims=True)
    w = jnp.where(d2 <= m3, 1.0 / jnp.maximum(d2, 1e-12), 0.0)
    wsum = jnp.sum(w, axis=1, keepdims=True)
    interp = jnp.dot(w, pf_ref[...], precision=_HIGHEST,
                     preferred_element_type=jnp.float32) / wsum
    y0 = (jnp.dot(q, w0c_ref[...], precision=_HIGHEST,
                  preferred_element_type=jnp.float32)
          + jnp.dot(of_ref[...], w0f_ref[...], precision=_HIGHEST,
                    preferred_element_type=jnp.float32)
          + jnp.dot(interp, w0i_ref[...], precision=_HIGHEST,
                    preferred_element_type=jnp.float32)
          + b0_ref[...])
    y0_ref[...] = y0

    @pl.when(i == 0)
    def _():
        st_ref[...] = jnp.zeros_like(st_ref)

    st_ref[0:1, :] = st_ref[0:1, :] + jnp.sum(y0, axis=0, keepdims=True)
    st_ref[1:2, :] = st_ref[1:2, :] + jnp.sum(y0 * y0, axis=0, keepdims=True)


def _make_pass_bc(n_rows, with_matmul):
    inv_n = 1.0 / n_rows

    def _pass_b(y_ref, st_ref, g_ref, be_ref, w1_ref, b1_ref, o_ref, st1_ref):
        i = pl.program_id(0)
        mean = st_ref[0:1, :] * inv_n
        var = st_ref[1:2, :] * inv_n - mean * mean
        scale = g_ref[...] * jax.lax.rsqrt(var + _EPS)
        shift = be_ref[...] - mean * scale
        h = jnp.maximum(y_ref[...] * scale + shift, 0.0)
        y1 = jnp.dot(h, w1_ref[...], precision=_HIGHEST,
                     preferred_element_type=jnp.float32) + b1_ref[...]
        o_ref[...] = y1

        @pl.when(i == 0)
        def _():
            st1_ref[...] = jnp.zeros_like(st1_ref)

        st1_ref[0:1, :] = st1_ref[0:1, :] + jnp.sum(y1, axis=0, keepdims=True)
        st1_ref[1:2, :] = st1_ref[1:2, :] + jnp.sum(y1 * y1, axis=0,
                                                    keepdims=True)

    def _pass_c(y_ref, st_ref, g_ref, be_ref, o_ref):
        mean = st_ref[0:1, :] * inv_n
        var = st_ref[1:2, :] * inv_n - mean * mean
        scale = g_ref[...] * jax.lax.rsqrt(var + _EPS)
        shift = be_ref[...] - mean * scale
        o_ref[...] = jnp.maximum(y_ref[...] * scale + shift, 0.0)

    return _pass_b if with_matmul else _pass_c


def kernel(prop_coords, prop_feats, orig_coords, orig_feats,
           W0, b0, g0, be0, W1, b1, g1, be1):
    n_l, _ = prop_coords.shape
    n_m = orig_coords.shape[0]
    f1 = prop_feats.shape[1]
    f2 = orig_feats.shape[1]
    h = W0.shape[1]
    tm = 256
    grid = n_m // tm
    f32 = jnp.float32

    qpad = jnp.pad(orig_coords, ((0, 0), (0, 5)))
    pt = jnp.pad(prop_coords, ((0, 0), (0, 5))).T
    w0c = jnp.pad(W0[:3], ((0, 5), (0, 0)))
    w0f = W0[3:3 + f2]
    w0i = W0[3 + f2:]
    b0r = b0.reshape(1, h)
    b1r = b1.reshape(1, h)
    g0r = g0.reshape(1, h)
    be0r = be0.reshape(1, h)
    g1r = g1.reshape(1, h)
    be1r = be1.reshape(1, h)

    row_spec = lambda w: pl.BlockSpec((tm, w), lambda i: (i, 0))
    full = lambda shape: pl.BlockSpec(shape, lambda i: (0, 0))

    y0, st0 = pl.pallas_call(
        _pass_a,
        grid=(grid,),
        in_specs=[row_spec(8), row_spec(f2), full((8, n_l)), full((n_l, f1)),
                  full((8, h)), full((f2, h)), full((f1, h)), full((1, h))],
        out_specs=[row_spec(h), full((8, h))],
        out_shape=[jax.ShapeDtypeStruct((n_m, h), f32),
                   jax.ShapeDtypeStruct((8, h), f32)],
    )(qpad, orig_feats, pt, prop_feats, w0c, w0f, w0i, b0r)

    y1, st1 = pl.pallas_call(
        _make_pass_bc(n_m, True),
        grid=(grid,),
        in_specs=[row_spec(h), full((8, h)), full((1, h)), full((1, h)),
                  full((h, h)), full((1, h))],
        out_specs=[row_spec(h), full((8, h))],
        out_shape=[jax.ShapeDtypeStruct((n_m, h), f32),
                   jax.ShapeDtypeStruct((8, h), f32)],
    )(y0, st0, g0r, be0r, W1, b1r)

    out = pl.pallas_call(
        _make_pass_bc(n_m, False),
        grid=(grid,),
        in_specs=[row_spec(h), full((8, h)), full((1, h)), full((1, h))],
        out_specs=row_spec(h),
        out_shape=jax.ShapeDtypeStruct((n_m, h), f32),
    )(y1, st1, g1r, be1r)

    return out
